# baseline (speedup 1.0000x reference)
import numpy as np
import jax
import jax.numpy as jnp
from jax.experimental import pallas as pl
from jax.experimental.pallas import tpu as pltpu

_B, _L, _K, _D, _T = 128, 8192, 4, 256, 100
_RP = 16
_TL = 8192
_NC = _TL // 2
_CW = 1024

_KS0 = np.uint32(0)
_KS1 = np.uint32(42)
_KS2 = np.uint32(0x1BD11BDA) ^ _KS0 ^ _KS1
_ROT = ((13, 15, 26, 6), (17, 29, 16, 24))
_TINY = np.float32(np.finfo(np.float32).tiny)


def _rotl(x, r):
    return (x << np.uint32(r)) | (x >> np.uint32(32 - r))


def _threefry_zero_hi(x1):
    ks = (_KS0, _KS1, _KS2)
    x0 = x1
    x1 = _rotl(x1, 13)
    x1 = x0 ^ x1
    for r in (15, 26, 6):
        x0 = x0 + x1
        x1 = _rotl(x1, r)
        x1 = x0 ^ x1
    x0 = x0 + ks[1]
    x1 = x1 + ks[2] + np.uint32(1)
    for i in range(1, 5):
        for r in _ROT[i % 2]:
            x0 = x0 + x1
            x1 = _rotl(x1, r)
            x1 = x0 ^ x1
        x0 = x0 + ks[(i + 1) % 3]
        x1 = x1 + ks[(i + 2) % 3] + np.uint32(i + 1)
    return x0 ^ x1


def _table_body(t2_ref, emb_ref, temb_ref, w_ref, b_ref, lp_ref):
    t2 = t2_ref[...]
    oh = (jax.lax.broadcasted_iota(jnp.int32, (_B, _T), 1) == t2)
    h_all = jax.lax.dot_general(oh.astype(jnp.float32), temb_ref[...],
                                (((1,), (0,)), ((), ())),
                                precision=jax.lax.Precision.HIGHEST,
                                preferred_element_type=jnp.float32)
    for v in range(_K):
        hv = jax.nn.gelu(h_all + emb_ref[v:v + 1, :])
        lg = jax.lax.dot_general(hv, w_ref[...], (((1,), (0,)), ((), ())),
                                 precision=jax.lax.Precision.HIGHEST,
                                 preferred_element_type=jnp.float32)
        lg = lg + b_ref[...]
        m = jnp.maximum(jnp.maximum(lg[:, 0:1], lg[:, 1:2]),
                        jnp.maximum(lg[:, 2:3], lg[:, 3:4]))
        e = jnp.exp(lg - m)
        p = e / (e[:, 0:1] + e[:, 1:2] + e[:, 2:3] + e[:, 3:4])
        lpv = jnp.log(p + np.float32(1e-20))
        gbest = p[:, 0:1]
        gidx = jnp.zeros((_B, 1), jnp.int32)
        for c in range(1, _K):
            hit = p[:, c:c + 1] > gbest
            gidx = jnp.where(hit, c, gidx)
            gbest = jnp.where(hit, p[:, c:c + 1], gbest)
        ci = jax.lax.broadcasted_iota(jnp.int32, (_B, _K), 1)
        det = jnp.where(ci == gidx, np.float32(0.0), np.float32(-1e30))
        lp_ref[:, 4 * v:4 * v + 4] = jnp.where(t2 == 0, det, lpv)


def _ddpm_kernel(x_ref, t2_ref, emb_ref, temb_ref, w_ref, b_ref, out_ref,
                 lp_ref):
    i = pl.program_id(0)

    @pl.when(i == 0)
    def _():
        _table_body(t2_ref, emb_ref, temb_ref, w_ref, b_ref, lp_ref)

    si = jax.lax.broadcasted_iota(jnp.int32, (8, _CW), 0)
    qi = jax.lax.broadcasted_iota(jnp.int32, (8, _CW), 1)
    pattern = (4 * qi + (si & 3) + (si >> 2) * (2 * _TL)).astype(jnp.uint32)
    idx0 = si & 3
    idxr1 = (si + 1) & 3
    shalf = si < 4

    for r in range(_RP):
        row = i * _RP + r
        lprow = lp_ref[pl.ds(row, 1), :]
        cm0 = idx0 == 0
        cm1 = idx0 == 1
        cm2 = idx0 == 2
        lv8 = [jnp.where(cm0, lprow[:, 4 * v:4 * v + 1],
               jnp.where(cm1, lprow[:, 4 * v + 1:4 * v + 2],
               jnp.where(cm2, lprow[:, 4 * v + 2:4 * v + 3],
                         lprow[:, 4 * v + 3:4 * v + 4])))
               for v in range(_K)]

        base = (i * _RP + r) * (_L * _K) + 42
        xv = x_ref[0, r:r + 1, :]
        for k in range(_NC // _CW):
            ctr = pattern + jnp.uint32(base + 4 * k * _CW)
            bits = _threefry_zero_hi(ctr)
            fb = jax.lax.bitcast_convert_type(
                (bits >> np.uint32(9)) | np.uint32(0x3F800000), jnp.float32)
            u = (fb - np.float32(1.0)) + _TINY
            g = -jnp.log(-jnp.log(u))

            vlo = jnp.broadcast_to(xv[:, k * _CW:(k + 1) * _CW], (8, _CW))
            vhi = jnp.broadcast_to(
                xv[:, _NC + k * _CW:_NC + (k + 1) * _CW], (8, _CW))
            v8 = jnp.where(shalf, vlo, vhi)
            score = g + jnp.where(v8 == 0, lv8[0],
                        jnp.where(v8 == 1, lv8[1],
                        jnp.where(v8 == 2, lv8[2], lv8[3])))

            s_r = jnp.roll(score, -1, axis=0)
            c1 = s_r > score
            s1 = jnp.where(c1, s_r, score)
            i1 = jnp.where(c1, idxr1, idx0)
            c2 = jnp.roll(s1, -2, axis=0) > s1
            i2 = jnp.where(c2, jnp.roll(i1, -2, axis=0), i1)
            out_ref[0, r:r + 1, pl.ds(k * _CW, _CW)] = i2[0:1, :]
            out_ref[0, r:r + 1, pl.ds(_NC + k * _CW, _CW)] = i2[4:5, :]


def kernel(x_t, t, emb, temb, W, b):
    x3 = x_t.reshape(_B // _RP, _RP, _L)
    b2 = b.reshape(1, _K)
    t2 = t.reshape(_B, 1)

    out = pl.pallas_call(
        _ddpm_kernel,
        grid=(_B // _RP,),
        in_specs=[
            pl.BlockSpec((1, _RP, _TL), lambda i: (i, 0, 0)),
            pl.BlockSpec((_B, 1), lambda i: (0, 0)),
            pl.BlockSpec((_K, _D), lambda i: (0, 0)),
            pl.BlockSpec((_T, _D), lambda i: (0, 0)),
            pl.BlockSpec((_D, _K), lambda i: (0, 0)),
            pl.BlockSpec((1, _K), lambda i: (0, 0)),
        ],
        out_specs=pl.BlockSpec((1, _RP, _TL), lambda i: (i, 0, 0)),
        out_shape=jax.ShapeDtypeStruct((_B // _RP, _RP, _L), jnp.int32),
        scratch_shapes=[pltpu.VMEM((_B, 4 * _K), jnp.float32)],
        compiler_params=pltpu.CompilerParams(
            dimension_semantics=("arbitrary",)),
    )(x3, t2, emb, temb, W, b2)
    return out.reshape(_B, _L)
